# baseline (speedup 1.0000x reference)
import functools

import jax
import jax.numpy as jnp
from jax import lax
from jax.experimental import pallas as pl
from jax.experimental.pallas import tpu as pltpu
from jax.experimental.pallas import tpu_sc as plsc

_N = 10000
_E = 160000
_DIM = 64
_SUB = 8
_NBONDS = 5
_RBDIM = 8
_CUTOFF = 5.0
_ZMAX = 86

_BE = 3200


def _ssp(x):
    return jax.nn.softplus(x) - jnp.log(2.0)


def _edge_msg_body(d_ref, sw_ref, bo_ref, hg_ref, w0_ref, out_ref):
    d = d_ref[...]
    sw = sw_ref[...]
    theta = (jnp.pi / _CUTOFF) * d
    s1 = jnp.sin(theta)
    c2 = 2.0 * jnp.cos(theta)
    pref = jnp.sqrt(2.0 / _CUTOFF) * sw / d
    sins = [s1, c2 * s1]
    for _ in range(_RBDIM - 2):
        sins.append(c2 * sins[-1] - sins[-2])
    m = hg_ref[...].T[_SUB:2 * _SUB, :]
    bo = bo_ref[...]
    pieces = []
    for n in range(_RBDIM):
        rn = pref * sins[n]
        for b in range(_NBONDS):
            g = rn * bo[b:b + 1, :]
            pieces.append(g * m)
    xij = jnp.concatenate(pieces, axis=0)
    out_ref[...] = jax.lax.dot_general(
        xij, w0_ref[...], (((0,), (0,)), ((), ())),
        preferred_element_type=jnp.float32)


def _edge_messages(dist2, sw2, boT, hg, w0p, blk0, nblk):
    return pl.pallas_call(
        _edge_msg_body,
        grid=(nblk,),
        in_specs=[
            pl.BlockSpec((1, _BE), lambda i: (0, i + blk0)),
            pl.BlockSpec((1, _BE), lambda i: (0, i + blk0)),
            pl.BlockSpec((_NBONDS, _BE), lambda i: (0, i + blk0)),
            pl.BlockSpec((_BE, 2 * _SUB), lambda i: (i + blk0, 0)),
            pl.BlockSpec((_RBDIM * _SUB * _NBONDS, _DIM), lambda i: (0, 0)),
        ],
        out_specs=pl.BlockSpec((_BE, _DIM), lambda i: (i, 0)),
        out_shape=jax.ShapeDtypeStruct((nblk * _BE, _DIM), jnp.float32),
    )(dist2, sw2, boT, hg, w0p)


_NC = 2
_NS = 16
_NW = _NC * _NS
_EW = _E // _NW
_CH = 40
_NPAD = 10240
_NPS = _NPAD // _NS


_SCH = 128
_NFULL = 39
_WBASE = _NFULL * _SCH
_XBASE = _WBASE * _NW
_NX = (_E - _XBASE) // _SCH


def _scat_body(e0, src_hbm, msg_hbm, out_hbm, idx0, idx1, rows0, rows1, zbuf,
               acc_sh, sem0, sem1):
    c = lax.axis_index("c")
    s = lax.axis_index("s")
    w = c * _NS + s

    def zfill(r, _):
        for k in range(_DIM // 16):
            zbuf[r, pl.ds(k * 16, 16)] = jnp.zeros((16,), jnp.float32)
        return 0

    lax.fori_loop(0, _NPS, zfill, 0)
    pltpu.sync_copy(zbuf, acc_sh.at[pl.ds(s * _NPS, _NPS)])
    plsc.subcore_barrier()

    base = w * _WBASE

    def start(g, idxv, rowsv, sem):
        eb = base + g * _SCH
        pltpu.async_copy(src_hbm.at[pl.ds(e0 + eb, _SCH)], idxv, sem)
        pltpu.async_copy(msg_hbm.at[pl.ds(eb, _SCH)], rowsv, sem)

    def fin(g, idxv, rowsv, sem):
        eb = base + g * _SCH
        pltpu.make_async_copy(src_hbm.at[pl.ds(e0 + eb, _SCH)], idxv,
                              sem).wait()
        pltpu.make_async_copy(msg_hbm.at[pl.ds(eb, _SCH)], rowsv, sem).wait()
        pltpu.sync_copy(rowsv, acc_sh.at[idxv], add=True)

    start(0, idx0, rows0, sem0)

    def body(i, _):
        g2 = 2 * i
        start(g2 + 1, idx1, rows1, sem1)
        fin(g2, idx0, rows0, sem0)
        start(g2 + 2, idx0, rows0, sem0)
        fin(g2 + 1, idx1, rows1, sem1)
        return 0

    lax.fori_loop(0, (_NFULL - 1) // 2, body, 0)
    fin(_NFULL - 1, idx0, rows0, sem0)

    @pl.when(w < _NX)
    def _extra():
        ebx = _XBASE + w * _SCH
        pltpu.sync_copy(src_hbm.at[pl.ds(e0 + ebx, _SCH)], idx0)
        pltpu.sync_copy(msg_hbm.at[pl.ds(ebx, _SCH)], rows0)
        pltpu.sync_copy(rows0, acc_sh.at[idx0], add=True)

    plsc.subcore_barrier()
    pltpu.sync_copy(acc_sh.at[pl.ds(s * _NPS, _NPS)], zbuf)
    pltpu.sync_copy(zbuf, out_hbm.at[w])


def _gath_body(pw, h_hbm, idx_hbm, out_hbm, idx_v, rows_v, sem):
    c = lax.axis_index("c")
    s = lax.axis_index("s")
    base = (c * _NS + s) * pw
    pltpu.sync_copy(idx_hbm.at[pl.ds(base, pw)], idx_v)
    pltpu.async_copy(h_hbm.at[idx_v], rows_v, sem).wait()
    pltpu.sync_copy(rows_v, out_hbm.at[pl.ds(base, pw)])


def _gather_sc(table, idx):
    b, d = idx.shape[0], table.shape[1]
    pw = b // _NW
    return pl.kernel(
        functools.partial(_gath_body, pw),
        compiler_params=pltpu.CompilerParams(use_tc_tiling_on_sc=False),
        out_type=jax.ShapeDtypeStruct((b, d), jnp.float32),
        mesh=plsc.VectorSubcoreMesh(core_axis_name="c", subcore_axis_name="s"),
        scratch_types=[
            pltpu.VMEM((pw,), jnp.int32),
            pltpu.VMEM((pw, d), jnp.float32),
            pltpu.SemaphoreType.DMA,
        ],
    )(table, idx)


def _gath2_body(pw, t1_hbm, t2_hbm, idx_hbm, out1_hbm, out2_hbm, idx_v,
                rows1_v, rows2_v, sem):
    c = lax.axis_index("c")
    s = lax.axis_index("s")
    base = (c * _NS + s) * pw
    pltpu.sync_copy(idx_hbm.at[pl.ds(base, pw)], idx_v)
    cp1 = pltpu.async_copy(t1_hbm.at[idx_v], rows1_v, sem)
    cp2 = pltpu.async_copy(t2_hbm.at[idx_v], rows2_v, sem)
    cp1.wait()
    cp2.wait()
    pltpu.sync_copy(rows1_v, out1_hbm.at[pl.ds(base, pw)])
    pltpu.sync_copy(rows2_v, out2_hbm.at[pl.ds(base, pw)])


def _gather2_sc(t1, t2, idx):
    b, d1, d2 = idx.shape[0], t1.shape[1], t2.shape[1]
    pw = b // _NW
    return pl.kernel(
        functools.partial(_gath2_body, pw),
        compiler_params=pltpu.CompilerParams(use_tc_tiling_on_sc=False),
        out_type=(jax.ShapeDtypeStruct((b, d1), jnp.float32),
                  jax.ShapeDtypeStruct((b, d2), jnp.float32)),
        mesh=plsc.VectorSubcoreMesh(core_axis_name="c", subcore_axis_name="s"),
        scratch_types=[
            pltpu.VMEM((pw,), jnp.int32),
            pltpu.VMEM((pw, d1), jnp.float32),
            pltpu.VMEM((pw, d2), jnp.float32),
            pltpu.SemaphoreType.DMA,
        ],
    )(t1, t2, idx)


def _segment_sum_sc(msg, edge_src, e0):
    out2 = pl.kernel(
        functools.partial(_scat_body, e0),
        compiler_params=pltpu.CompilerParams(use_tc_tiling_on_sc=False),
        out_type=jax.ShapeDtypeStruct((_NW, _NPS, _DIM), jnp.float32),
        mesh=plsc.VectorSubcoreMesh(core_axis_name="c", subcore_axis_name="s"),
        scratch_types=[
            pltpu.VMEM((_SCH,), jnp.int32),
            pltpu.VMEM((_SCH,), jnp.int32),
            pltpu.VMEM((_SCH, _DIM), jnp.float32),
            pltpu.VMEM((_SCH, _DIM), jnp.float32),
            pltpu.VMEM((_NPS, _DIM), jnp.float32),
            pltpu.VMEM_SHARED((_NPAD, _DIM), jnp.float32),
            pltpu.SemaphoreType.DMA,
            pltpu.SemaphoreType.DMA,
        ],
    )(edge_src, msg)
    return out2.reshape(_NC, _NPAD, _DIM)


_BN = 2048


def _ssp_p(x):
    return (jnp.maximum(x, 0.0) + jnp.log(1.0 + jnp.exp(-jnp.abs(x)))
            - jnp.log(2.0))


def _node_first_body(xi_ref, lw_ref, lb_ref, ho_ref):
    ho_ref[...] = jnp.dot(xi_ref[...], lw_ref[...],
                          preferred_element_type=jnp.float32) + lb_ref[...]


def _node_first(xi, lw, lb):
    return pl.pallas_call(
        _node_first_body,
        grid=(_NPAD // _BN,),
        in_specs=[
            pl.BlockSpec((_BN, _DIM), lambda i: (i, 0)),
            pl.BlockSpec((_DIM, 2 * _SUB), lambda i: (0, 0)),
            pl.BlockSpec((1, 2 * _SUB), lambda i: (0, 0)),
        ],
        out_specs=pl.BlockSpec((_BN, 2 * _SUB), lambda i: (i, 0)),
        out_shape=jax.ShapeDtypeStruct((_NPAD, 2 * _SUB), jnp.float32),
    )(xi, lw, lb.reshape(1, 2 * _SUB))


def _node_post_body(has_next, xi_ref, h_ref, agg_ref, w0t_ref,
                    b0_ref, w1_ref, b1_ref, w2_ref, b2_ref, *rest):
    if has_next:
        lw_ref, lb_ref, xo_ref, ho_ref = rest
    else:
        xo_ref, = rest
    agg = agg_ref[0] + agg_ref[1]
    si = h_ref[:, :_SUB]
    dxi = agg + jnp.dot(si, w0t_ref[...],
                        preferred_element_type=jnp.float32) + b0_ref[...]
    dxi = _ssp_p(dxi)
    dxi = _ssp_p(jnp.dot(dxi, w1_ref[...],
                         preferred_element_type=jnp.float32) + b1_ref[...])
    dxi = jnp.dot(dxi, w2_ref[...],
                  preferred_element_type=jnp.float32) + b2_ref[...]
    xn = xi_ref[...] + dxi
    xo_ref[...] = xn
    if has_next:
        ho_ref[...] = jnp.dot(xn, lw_ref[...],
                              preferred_element_type=jnp.float32) + lb_ref[...]


def _node_post(xi, h, agg2, w0t, b0, w1, b1, w2, b2, lw=None, lb=None):
    has_next = lw is not None
    in_specs = [
        pl.BlockSpec((_BN, _DIM), lambda i: (i, 0)),
        pl.BlockSpec((_BN, 2 * _SUB), lambda i: (i, 0)),
        pl.BlockSpec((_NC, _BN, _DIM), lambda i: (0, i, 0)),
        pl.BlockSpec((_SUB, _DIM), lambda i: (0, 0)),
        pl.BlockSpec((1, _DIM), lambda i: (0, 0)),
        pl.BlockSpec((_DIM, _DIM), lambda i: (0, 0)),
        pl.BlockSpec((1, _DIM), lambda i: (0, 0)),
        pl.BlockSpec((_DIM, _DIM), lambda i: (0, 0)),
        pl.BlockSpec((1, _DIM), lambda i: (0, 0)),
    ]
    args = [xi, h, agg2, w0t, b0.reshape(1, _DIM), w1,
            b1.reshape(1, _DIM), w2, b2.reshape(1, _DIM)]
    out_specs = [pl.BlockSpec((_BN, _DIM), lambda i: (i, 0))]
    out_shape = [jax.ShapeDtypeStruct((_NPAD, _DIM), jnp.float32)]
    if has_next:
        in_specs += [pl.BlockSpec((_DIM, 2 * _SUB), lambda i: (0, 0)),
                     pl.BlockSpec((1, 2 * _SUB), lambda i: (0, 0))]
        args += [lw, lb.reshape(1, 2 * _SUB)]
        out_specs += [pl.BlockSpec((_BN, 2 * _SUB), lambda i: (i, 0))]
        out_shape += [jax.ShapeDtypeStruct((_NPAD, 2 * _SUB), jnp.float32)]
    res = pl.pallas_call(
        functools.partial(_node_post_body, has_next),
        grid=(_NPAD // _BN,),
        in_specs=in_specs,
        out_specs=out_specs,
        out_shape=out_shape,
    )(*args)
    return res if has_next else (res[0],)


def kernel(species, edge_src, edge_dst, distances, switch, bond_order, params):
    p = params
    nfc = _RBDIM * _SUB * _NBONDS
    sp_pad = jnp.pad(species.astype(jnp.int32), (0, _NPAD - _N))
    t2 = p['W_species'] @ p['lin0_W'] + p['lin0_b'][None, :]
    xi, h = _gather2_sc(p['W_species'], t2, sp_pad)
    dist2 = distances.reshape(1, _E)
    sw2 = switch.reshape(1, _E)
    boT = bond_order.T
    nblk = _E // _BE
    for l in range(3):
        w0 = p[f'fc{l}_W0']
        w0p = (w0[:nfc].reshape(_RBDIM, _SUB, _NBONDS, _DIM)
               .transpose(0, 2, 1, 3).reshape(nfc, _DIM))
        hg = _gather_sc(h, edge_dst)
        msg = _edge_messages(dist2, sw2, boT, hg, w0p, 0, nblk)
        agg2 = _segment_sum_sc(msg, edge_src, 0)
        nxt = (p[f'lin{l + 1}_W'], p[f'lin{l + 1}_b']) if l < 2 else (None,
                                                                      None)
        out = _node_post(xi, h, agg2, w0[nfc:], p[f'fc{l}_b0'],
                         p[f'fc{l}_W1'], p[f'fc{l}_b1'],
                         p[f'fc{l}_W2'], p[f'fc{l}_b2'], *nxt)
        if l < 2:
            xi, h = out
        else:
            xi = out[0]
    return xi[:_N]


# speedup vs baseline: 1.0163x; 1.0163x over previous
import functools

import jax
import jax.numpy as jnp
from jax import lax
from jax.experimental import pallas as pl
from jax.experimental.pallas import tpu as pltpu
from jax.experimental.pallas import tpu_sc as plsc

_N = 10000
_E = 160000
_DIM = 64
_SUB = 8
_NBONDS = 5
_RBDIM = 8
_CUTOFF = 5.0
_ZMAX = 86

_BE = 3200


def _ssp(x):
    return jax.nn.softplus(x) - jnp.log(2.0)


def _edge_msg_body(d_ref, sw_ref, bo_ref, hg_ref, w0_ref, out_ref):
    d = d_ref[...]
    sw = sw_ref[...]
    theta = (jnp.pi / _CUTOFF) * d
    s1 = jnp.sin(theta)
    c2 = 2.0 * jnp.cos(theta)
    pref = jnp.sqrt(2.0 / _CUTOFF) * sw / d
    sins = [s1, c2 * s1]
    for _ in range(_RBDIM - 2):
        sins.append(c2 * sins[-1] - sins[-2])
    m = hg_ref[...].T[_SUB:2 * _SUB, :]
    bo = bo_ref[...]
    pieces = []
    for n in range(_RBDIM):
        rn = pref * sins[n]
        for b in range(_NBONDS):
            g = rn * bo[b:b + 1, :]
            pieces.append(g * m)
    xij = jnp.concatenate(pieces, axis=0)
    out_ref[...] = jax.lax.dot_general(
        xij, w0_ref[...], (((0,), (0,)), ((), ())),
        preferred_element_type=jnp.float32)


def _edge_messages(dist2, sw2, boT, hg, w0p, blk0, nblk):
    return pl.pallas_call(
        _edge_msg_body,
        grid=(nblk,),
        in_specs=[
            pl.BlockSpec((1, _BE), lambda i: (0, i + blk0)),
            pl.BlockSpec((1, _BE), lambda i: (0, i + blk0)),
            pl.BlockSpec((_NBONDS, _BE), lambda i: (0, i + blk0)),
            pl.BlockSpec((_BE, 2 * _SUB), lambda i: (i + blk0, 0)),
            pl.BlockSpec((_RBDIM * _SUB * _NBONDS, _DIM), lambda i: (0, 0)),
        ],
        out_specs=pl.BlockSpec((_BE, _DIM), lambda i: (i, 0)),
        out_shape=jax.ShapeDtypeStruct((nblk * _BE, _DIM), jnp.float32),
    )(dist2, sw2, boT, hg, w0p)


_NC = 2
_NS = 16
_NW = _NC * _NS
_EW = _E // _NW
_CH = 40
_NPAD = 10240
_NPS = _NPAD // _NS


_SCH = 128
_EH = _E // 2
_NFULL = 19
_WBASE = _NFULL * _SCH
_XBASE = _WBASE * _NW
_NX = (_EH - _XBASE) // _SCH


def _scat_body(e0, src_hbm, msg_hbm, out_hbm, idx0, idx1, rows0, rows1, zbuf,
               acc_sh, sem0, sem1):
    c = lax.axis_index("c")
    s = lax.axis_index("s")
    w = c * _NS + s

    def zfill(r, _):
        for k in range(_DIM // 16):
            zbuf[r, pl.ds(k * 16, 16)] = jnp.zeros((16,), jnp.float32)
        return 0

    lax.fori_loop(0, _NPS, zfill, 0)
    pltpu.sync_copy(zbuf, acc_sh.at[pl.ds(s * _NPS, _NPS)])
    plsc.subcore_barrier()

    base = w * _WBASE

    def start(g, idxv, rowsv, sem):
        eb = base + g * _SCH
        pltpu.async_copy(src_hbm.at[pl.ds(e0 + eb, _SCH)], idxv, sem)
        pltpu.async_copy(msg_hbm.at[pl.ds(eb, _SCH)], rowsv, sem)

    def fin(g, idxv, rowsv, sem):
        eb = base + g * _SCH
        pltpu.make_async_copy(src_hbm.at[pl.ds(e0 + eb, _SCH)], idxv,
                              sem).wait()
        pltpu.make_async_copy(msg_hbm.at[pl.ds(eb, _SCH)], rowsv, sem).wait()
        pltpu.sync_copy(rowsv, acc_sh.at[idxv], add=True)

    start(0, idx0, rows0, sem0)

    def body(i, _):
        g2 = 2 * i
        start(g2 + 1, idx1, rows1, sem1)
        fin(g2, idx0, rows0, sem0)
        start(g2 + 2, idx0, rows0, sem0)
        fin(g2 + 1, idx1, rows1, sem1)
        return 0

    lax.fori_loop(0, (_NFULL - 1) // 2, body, 0)
    fin(_NFULL - 1, idx0, rows0, sem0)

    @pl.when(w < _NX)
    def _extra():
        ebx = _XBASE + w * _SCH
        pltpu.sync_copy(src_hbm.at[pl.ds(e0 + ebx, _SCH)], idx0)
        pltpu.sync_copy(msg_hbm.at[pl.ds(ebx, _SCH)], rows0)
        pltpu.sync_copy(rows0, acc_sh.at[idx0], add=True)

    plsc.subcore_barrier()
    pltpu.sync_copy(acc_sh.at[pl.ds(s * _NPS, _NPS)], zbuf)
    pltpu.sync_copy(zbuf, out_hbm.at[w])


def _gath_body(pw, h_hbm, idx_hbm, out_hbm, idx_v, rows_v, sem):
    c = lax.axis_index("c")
    s = lax.axis_index("s")
    base = (c * _NS + s) * pw
    pltpu.sync_copy(idx_hbm.at[pl.ds(base, pw)], idx_v)
    pltpu.async_copy(h_hbm.at[idx_v], rows_v, sem).wait()
    pltpu.sync_copy(rows_v, out_hbm.at[pl.ds(base, pw)])


def _gather_sc(table, idx):
    b, d = idx.shape[0], table.shape[1]
    pw = b // _NW
    return pl.kernel(
        functools.partial(_gath_body, pw),
        compiler_params=pltpu.CompilerParams(use_tc_tiling_on_sc=False),
        out_type=jax.ShapeDtypeStruct((b, d), jnp.float32),
        mesh=plsc.VectorSubcoreMesh(core_axis_name="c", subcore_axis_name="s"),
        scratch_types=[
            pltpu.VMEM((pw,), jnp.int32),
            pltpu.VMEM((pw, d), jnp.float32),
            pltpu.SemaphoreType.DMA,
        ],
    )(table, idx)


def _gath2_body(pw, t1_hbm, t2_hbm, idx_hbm, out1_hbm, out2_hbm, idx_v,
                rows1_v, rows2_v, sem):
    c = lax.axis_index("c")
    s = lax.axis_index("s")
    base = (c * _NS + s) * pw
    pltpu.sync_copy(idx_hbm.at[pl.ds(base, pw)], idx_v)
    cp1 = pltpu.async_copy(t1_hbm.at[idx_v], rows1_v, sem)
    cp2 = pltpu.async_copy(t2_hbm.at[idx_v], rows2_v, sem)
    cp1.wait()
    cp2.wait()
    pltpu.sync_copy(rows1_v, out1_hbm.at[pl.ds(base, pw)])
    pltpu.sync_copy(rows2_v, out2_hbm.at[pl.ds(base, pw)])


def _gather2_sc(t1, t2, idx):
    b, d1, d2 = idx.shape[0], t1.shape[1], t2.shape[1]
    pw = b // _NW
    return pl.kernel(
        functools.partial(_gath2_body, pw),
        compiler_params=pltpu.CompilerParams(use_tc_tiling_on_sc=False),
        out_type=(jax.ShapeDtypeStruct((b, d1), jnp.float32),
                  jax.ShapeDtypeStruct((b, d2), jnp.float32)),
        mesh=plsc.VectorSubcoreMesh(core_axis_name="c", subcore_axis_name="s"),
        scratch_types=[
            pltpu.VMEM((pw,), jnp.int32),
            pltpu.VMEM((pw, d1), jnp.float32),
            pltpu.VMEM((pw, d2), jnp.float32),
            pltpu.SemaphoreType.DMA,
        ],
    )(t1, t2, idx)


def _segment_sum_sc(msg, edge_src, e0):
    out2 = pl.kernel(
        functools.partial(_scat_body, e0),
        compiler_params=pltpu.CompilerParams(use_tc_tiling_on_sc=False),
        out_type=jax.ShapeDtypeStruct((_NW, _NPS, _DIM), jnp.float32),
        mesh=plsc.VectorSubcoreMesh(core_axis_name="c", subcore_axis_name="s"),
        scratch_types=[
            pltpu.VMEM((_SCH,), jnp.int32),
            pltpu.VMEM((_SCH,), jnp.int32),
            pltpu.VMEM((_SCH, _DIM), jnp.float32),
            pltpu.VMEM((_SCH, _DIM), jnp.float32),
            pltpu.VMEM((_NPS, _DIM), jnp.float32),
            pltpu.VMEM_SHARED((_NPAD, _DIM), jnp.float32),
            pltpu.SemaphoreType.DMA,
            pltpu.SemaphoreType.DMA,
        ],
    )(edge_src, msg)
    return out2.reshape(_NC, _NPAD, _DIM)


_BN = 2048


def _ssp_p(x):
    return (jnp.maximum(x, 0.0) + jnp.log(1.0 + jnp.exp(-jnp.abs(x)))
            - jnp.log(2.0))


def _node_first_body(xi_ref, lw_ref, lb_ref, ho_ref):
    ho_ref[...] = jnp.dot(xi_ref[...], lw_ref[...],
                          preferred_element_type=jnp.float32) + lb_ref[...]


def _node_first(xi, lw, lb):
    return pl.pallas_call(
        _node_first_body,
        grid=(_NPAD // _BN,),
        in_specs=[
            pl.BlockSpec((_BN, _DIM), lambda i: (i, 0)),
            pl.BlockSpec((_DIM, 2 * _SUB), lambda i: (0, 0)),
            pl.BlockSpec((1, 2 * _SUB), lambda i: (0, 0)),
        ],
        out_specs=pl.BlockSpec((_BN, 2 * _SUB), lambda i: (i, 0)),
        out_shape=jax.ShapeDtypeStruct((_NPAD, 2 * _SUB), jnp.float32),
    )(xi, lw, lb.reshape(1, 2 * _SUB))


def _node_post_body(has_next, xi_ref, h_ref, agg_ref, aggb_ref, w0t_ref,
                    b0_ref, w1_ref, b1_ref, w2_ref, b2_ref, *rest):
    if has_next:
        lw_ref, lb_ref, xo_ref, ho_ref = rest
    else:
        xo_ref, = rest
    agg = (agg_ref[0] + agg_ref[1]) + (aggb_ref[0] + aggb_ref[1])
    si = h_ref[:, :_SUB]
    dxi = agg + jnp.dot(si, w0t_ref[...],
                        preferred_element_type=jnp.float32) + b0_ref[...]
    dxi = _ssp_p(dxi)
    dxi = _ssp_p(jnp.dot(dxi, w1_ref[...],
                         preferred_element_type=jnp.float32) + b1_ref[...])
    dxi = jnp.dot(dxi, w2_ref[...],
                  preferred_element_type=jnp.float32) + b2_ref[...]
    xn = xi_ref[...] + dxi
    xo_ref[...] = xn
    if has_next:
        ho_ref[...] = jnp.dot(xn, lw_ref[...],
                              preferred_element_type=jnp.float32) + lb_ref[...]


def _node_post(xi, h, agg2, agg2b, w0t, b0, w1, b1, w2, b2, lw=None, lb=None):
    has_next = lw is not None
    in_specs = [
        pl.BlockSpec((_BN, _DIM), lambda i: (i, 0)),
        pl.BlockSpec((_BN, 2 * _SUB), lambda i: (i, 0)),
        pl.BlockSpec((_NC, _BN, _DIM), lambda i: (0, i, 0)),
        pl.BlockSpec((_NC, _BN, _DIM), lambda i: (0, i, 0)),
        pl.BlockSpec((_SUB, _DIM), lambda i: (0, 0)),
        pl.BlockSpec((1, _DIM), lambda i: (0, 0)),
        pl.BlockSpec((_DIM, _DIM), lambda i: (0, 0)),
        pl.BlockSpec((1, _DIM), lambda i: (0, 0)),
        pl.BlockSpec((_DIM, _DIM), lambda i: (0, 0)),
        pl.BlockSpec((1, _DIM), lambda i: (0, 0)),
    ]
    args = [xi, h, agg2, agg2b, w0t, b0.reshape(1, _DIM), w1,
            b1.reshape(1, _DIM), w2, b2.reshape(1, _DIM)]
    out_specs = [pl.BlockSpec((_BN, _DIM), lambda i: (i, 0))]
    out_shape = [jax.ShapeDtypeStruct((_NPAD, _DIM), jnp.float32)]
    if has_next:
        in_specs += [pl.BlockSpec((_DIM, 2 * _SUB), lambda i: (0, 0)),
                     pl.BlockSpec((1, 2 * _SUB), lambda i: (0, 0))]
        args += [lw, lb.reshape(1, 2 * _SUB)]
        out_specs += [pl.BlockSpec((_BN, 2 * _SUB), lambda i: (i, 0))]
        out_shape += [jax.ShapeDtypeStruct((_NPAD, 2 * _SUB), jnp.float32)]
    res = pl.pallas_call(
        functools.partial(_node_post_body, has_next),
        grid=(_NPAD // _BN,),
        in_specs=in_specs,
        out_specs=out_specs,
        out_shape=out_shape,
    )(*args)
    return res if has_next else (res[0],)


def kernel(species, edge_src, edge_dst, distances, switch, bond_order, params):
    p = params
    nfc = _RBDIM * _SUB * _NBONDS
    sp_pad = jnp.pad(species.astype(jnp.int32), (0, _NPAD - _N))
    t2 = p['W_species'] @ p['lin0_W'] + p['lin0_b'][None, :]
    xi, h = _gather2_sc(p['W_species'], t2, sp_pad)
    dist2 = distances.reshape(1, _E)
    sw2 = switch.reshape(1, _E)
    boT = bond_order.T
    nblk_h = _EH // _BE
    for l in range(3):
        w0 = p[f'fc{l}_W0']
        w0p = (w0[:nfc].reshape(_RBDIM, _SUB, _NBONDS, _DIM)
               .transpose(0, 2, 1, 3).reshape(nfc, _DIM))
        hg = _gather_sc(h, edge_dst)
        msga = _edge_messages(dist2, sw2, boT, hg, w0p, 0, nblk_h)
        agg2a = _segment_sum_sc(msga, edge_src, 0)
        msgb = _edge_messages(dist2, sw2, boT, hg, w0p, nblk_h, nblk_h)
        agg2b = _segment_sum_sc(msgb, edge_src, _EH)
        nxt = (p[f'lin{l + 1}_W'], p[f'lin{l + 1}_b']) if l < 2 else (None,
                                                                      None)
        out = _node_post(xi, h, agg2a, agg2b, w0[nfc:], p[f'fc{l}_b0'],
                         p[f'fc{l}_W1'], p[f'fc{l}_b1'],
                         p[f'fc{l}_W2'], p[f'fc{l}_b2'], *nxt)
        if l < 2:
            xi, h = out
        else:
            xi = out[0]
    return xi[:_N]


# speedup vs baseline: 1.0167x; 1.0005x over previous
import functools

import jax
import jax.numpy as jnp
from jax import lax
from jax.experimental import pallas as pl
from jax.experimental.pallas import tpu as pltpu
from jax.experimental.pallas import tpu_sc as plsc

_N = 10000
_E = 160000
_DIM = 64
_SUB = 8
_NBONDS = 5
_RBDIM = 8
_CUTOFF = 5.0
_ZMAX = 86

_BE = 3200


def _edge_msg_body(d_ref, sw_ref, bo_ref, hg_ref, w0_ref, out_ref):
    d = d_ref[...]
    sw = sw_ref[...]
    theta = (jnp.pi / _CUTOFF) * d
    s1 = jnp.sin(theta)
    c2 = 2.0 * jnp.cos(theta)
    pref = jnp.sqrt(2.0 / _CUTOFF) * sw / d
    sins = [s1, c2 * s1]
    for _ in range(_RBDIM - 2):
        sins.append(c2 * sins[-1] - sins[-2])
    m = hg_ref[...].T[_SUB:2 * _SUB, :]
    bo = bo_ref[...]
    pieces = []
    for n in range(_RBDIM):
        rn = pref * sins[n]
        for b in range(_NBONDS):
            g = rn * bo[b:b + 1, :]
            pieces.append(g * m)
    xij = jnp.concatenate(pieces, axis=0)
    out_ref[...] = jax.lax.dot_general(
        xij, w0_ref[...], (((0,), (0,)), ((), ())),
        preferred_element_type=jnp.float32)


def _edge_messages(dist2, sw2, boT, hg, w0p, blk0, nblk):
    return pl.pallas_call(
        _edge_msg_body,
        grid=(nblk,),
        in_specs=[
            pl.BlockSpec((1, _BE), lambda i: (0, i + blk0)),
            pl.BlockSpec((1, _BE), lambda i: (0, i + blk0)),
            pl.BlockSpec((_NBONDS, _BE), lambda i: (0, i + blk0)),
            pl.BlockSpec((_BE, 2 * _SUB), lambda i: (i + blk0, 0)),
            pl.BlockSpec((_RBDIM * _SUB * _NBONDS, _DIM), lambda i: (0, 0)),
        ],
        out_specs=pl.BlockSpec((_BE, _DIM), lambda i: (i, 0)),
        out_shape=jax.ShapeDtypeStruct((nblk * _BE, _DIM), jnp.float32),
    )(dist2, sw2, boT, hg, w0p)


_NC = 2
_NS = 16
_NW = _NC * _NS
_NPAD = 10240
_NPS = _NPAD // _NS


_SCH = 128
_EH = _E // 2
_NFULL = 19
_WBASE = _NFULL * _SCH
_XBASE = _WBASE * _NW
_NX = (_EH - _XBASE) // _SCH


def _scat_body(e0, src_hbm, msg_hbm, out_hbm, idx0, idx1, rows0, rows1, zbuf,
               acc_sh, sem0, sem1):
    c = lax.axis_index("c")
    s = lax.axis_index("s")
    w = c * _NS + s

    def zfill(r, _):
        for k in range(_DIM // 16):
            zbuf[r, pl.ds(k * 16, 16)] = jnp.zeros((16,), jnp.float32)
        return 0

    lax.fori_loop(0, _NPS, zfill, 0)
    pltpu.sync_copy(zbuf, acc_sh.at[pl.ds(s * _NPS, _NPS)])
    plsc.subcore_barrier()

    base = w * _WBASE

    def start(g, idxv, rowsv, sem):
        eb = base + g * _SCH
        pltpu.async_copy(src_hbm.at[pl.ds(e0 + eb, _SCH)], idxv, sem)
        pltpu.async_copy(msg_hbm.at[pl.ds(eb, _SCH)], rowsv, sem)

    def fin(g, idxv, rowsv, sem):
        eb = base + g * _SCH
        pltpu.make_async_copy(src_hbm.at[pl.ds(e0 + eb, _SCH)], idxv,
                              sem).wait()
        pltpu.make_async_copy(msg_hbm.at[pl.ds(eb, _SCH)], rowsv, sem).wait()
        pltpu.sync_copy(rowsv, acc_sh.at[idxv], add=True)

    start(0, idx0, rows0, sem0)

    def body(i, _):
        g2 = 2 * i
        start(g2 + 1, idx1, rows1, sem1)
        fin(g2, idx0, rows0, sem0)
        start(g2 + 2, idx0, rows0, sem0)
        fin(g2 + 1, idx1, rows1, sem1)
        return 0

    lax.fori_loop(0, (_NFULL - 1) // 2, body, 0)
    fin(_NFULL - 1, idx0, rows0, sem0)

    @pl.when(w < _NX)
    def _extra():
        ebx = _XBASE + w * _SCH
        pltpu.sync_copy(src_hbm.at[pl.ds(e0 + ebx, _SCH)], idx0)
        pltpu.sync_copy(msg_hbm.at[pl.ds(ebx, _SCH)], rows0)
        pltpu.sync_copy(rows0, acc_sh.at[idx0], add=True)

    plsc.subcore_barrier()
    pltpu.sync_copy(acc_sh.at[pl.ds(s * _NPS, _NPS)], zbuf)
    pltpu.sync_copy(zbuf, out_hbm.at[w])


def _gath_body(pw, h_hbm, idx_hbm, out_hbm, idx_v, rows_v, sem):
    c = lax.axis_index("c")
    s = lax.axis_index("s")
    base = (c * _NS + s) * pw
    pltpu.sync_copy(idx_hbm.at[pl.ds(base, pw)], idx_v)
    pltpu.async_copy(h_hbm.at[idx_v], rows_v, sem).wait()
    pltpu.sync_copy(rows_v, out_hbm.at[pl.ds(base, pw)])


def _gather_sc(table, idx):
    b, d = idx.shape[0], table.shape[1]
    pw = b // _NW
    return pl.kernel(
        functools.partial(_gath_body, pw),
        compiler_params=pltpu.CompilerParams(use_tc_tiling_on_sc=False),
        out_type=jax.ShapeDtypeStruct((b, d), jnp.float32),
        mesh=plsc.VectorSubcoreMesh(core_axis_name="c", subcore_axis_name="s"),
        scratch_types=[
            pltpu.VMEM((pw,), jnp.int32),
            pltpu.VMEM((pw, d), jnp.float32),
            pltpu.SemaphoreType.DMA,
        ],
    )(table, idx)


def _gath2_body(pw, t1_hbm, t2_hbm, idx_hbm, out1_hbm, out2_hbm, idx_v,
                rows1_v, rows2_v, sem):
    c = lax.axis_index("c")
    s = lax.axis_index("s")
    base = (c * _NS + s) * pw
    pltpu.sync_copy(idx_hbm.at[pl.ds(base, pw)], idx_v)
    cp1 = pltpu.async_copy(t1_hbm.at[idx_v], rows1_v, sem)
    cp2 = pltpu.async_copy(t2_hbm.at[idx_v], rows2_v, sem)
    cp1.wait()
    cp2.wait()
    pltpu.sync_copy(rows1_v, out1_hbm.at[pl.ds(base, pw)])
    pltpu.sync_copy(rows2_v, out2_hbm.at[pl.ds(base, pw)])


def _gather2_sc(t1, t2, idx):
    b, d1, d2 = idx.shape[0], t1.shape[1], t2.shape[1]
    pw = b // _NW
    return pl.kernel(
        functools.partial(_gath2_body, pw),
        compiler_params=pltpu.CompilerParams(use_tc_tiling_on_sc=False),
        out_type=(jax.ShapeDtypeStruct((b, d1), jnp.float32),
                  jax.ShapeDtypeStruct((b, d2), jnp.float32)),
        mesh=plsc.VectorSubcoreMesh(core_axis_name="c", subcore_axis_name="s"),
        scratch_types=[
            pltpu.VMEM((pw,), jnp.int32),
            pltpu.VMEM((pw, d1), jnp.float32),
            pltpu.VMEM((pw, d2), jnp.float32),
            pltpu.SemaphoreType.DMA,
        ],
    )(t1, t2, idx)


def _segment_sum_sc(msg, edge_src, e0):
    out2 = pl.kernel(
        functools.partial(_scat_body, e0),
        compiler_params=pltpu.CompilerParams(use_tc_tiling_on_sc=False),
        out_type=jax.ShapeDtypeStruct((_NW, _NPS, _DIM), jnp.float32),
        mesh=plsc.VectorSubcoreMesh(core_axis_name="c", subcore_axis_name="s"),
        scratch_types=[
            pltpu.VMEM((_SCH,), jnp.int32),
            pltpu.VMEM((_SCH,), jnp.int32),
            pltpu.VMEM((_SCH, _DIM), jnp.float32),
            pltpu.VMEM((_SCH, _DIM), jnp.float32),
            pltpu.VMEM((_NPS, _DIM), jnp.float32),
            pltpu.VMEM_SHARED((_NPAD, _DIM), jnp.float32),
            pltpu.SemaphoreType.DMA,
            pltpu.SemaphoreType.DMA,
        ],
    )(edge_src, msg)
    return out2.reshape(_NC, _NPAD, _DIM)


_BN = 2048


def _ssp_p(x):
    return (jnp.maximum(x, 0.0) + jnp.log(1.0 + jnp.exp(-jnp.abs(x)))
            - jnp.log(2.0))


def _node_post_body(has_next, xi_ref, h_ref, agg_ref, aggb_ref, w0t_ref,
                    b0_ref, w1_ref, b1_ref, w2_ref, b2_ref, *rest):
    if has_next:
        lw_ref, lb_ref, xo_ref, ho_ref = rest
    else:
        xo_ref, = rest
    agg = (agg_ref[0] + agg_ref[1]) + (aggb_ref[0] + aggb_ref[1])
    si = h_ref[:, :_SUB]
    dxi = agg + jnp.dot(si, w0t_ref[...],
                        preferred_element_type=jnp.float32) + b0_ref[...]
    dxi = _ssp_p(dxi)
    dxi = _ssp_p(jnp.dot(dxi, w1_ref[...],
                         preferred_element_type=jnp.float32) + b1_ref[...])
    dxi = jnp.dot(dxi, w2_ref[...],
                  preferred_element_type=jnp.float32) + b2_ref[...]
    xn = xi_ref[...] + dxi
    xo_ref[...] = xn
    if has_next:
        ho_ref[...] = jnp.dot(xn, lw_ref[...],
                              preferred_element_type=jnp.float32) + lb_ref[...]


def _node_post(xi, h, agg2, agg2b, w0t, b0, w1, b1, w2, b2, lw=None, lb=None):
    has_next = lw is not None
    in_specs = [
        pl.BlockSpec((_BN, _DIM), lambda i: (i, 0)),
        pl.BlockSpec((_BN, 2 * _SUB), lambda i: (i, 0)),
        pl.BlockSpec((_NC, _BN, _DIM), lambda i: (0, i, 0)),
        pl.BlockSpec((_NC, _BN, _DIM), lambda i: (0, i, 0)),
        pl.BlockSpec((_SUB, _DIM), lambda i: (0, 0)),
        pl.BlockSpec((1, _DIM), lambda i: (0, 0)),
        pl.BlockSpec((_DIM, _DIM), lambda i: (0, 0)),
        pl.BlockSpec((1, _DIM), lambda i: (0, 0)),
        pl.BlockSpec((_DIM, _DIM), lambda i: (0, 0)),
        pl.BlockSpec((1, _DIM), lambda i: (0, 0)),
    ]
    args = [xi, h, agg2, agg2b, w0t, b0.reshape(1, _DIM), w1,
            b1.reshape(1, _DIM), w2, b2.reshape(1, _DIM)]
    out_specs = [pl.BlockSpec((_BN, _DIM), lambda i: (i, 0))]
    out_shape = [jax.ShapeDtypeStruct((_NPAD, _DIM), jnp.float32)]
    if has_next:
        in_specs += [pl.BlockSpec((_DIM, 2 * _SUB), lambda i: (0, 0)),
                     pl.BlockSpec((1, 2 * _SUB), lambda i: (0, 0))]
        args += [lw, lb.reshape(1, 2 * _SUB)]
        out_specs += [pl.BlockSpec((_BN, 2 * _SUB), lambda i: (i, 0))]
        out_shape += [jax.ShapeDtypeStruct((_NPAD, 2 * _SUB), jnp.float32)]
    res = pl.pallas_call(
        functools.partial(_node_post_body, has_next),
        grid=(_NPAD // _BN,),
        in_specs=in_specs,
        out_specs=out_specs,
        out_shape=out_shape,
    )(*args)
    return res if has_next else (res[0],)


def kernel(species, edge_src, edge_dst, distances, switch, bond_order, params):
    p = params
    nfc = _RBDIM * _SUB * _NBONDS
    sp_pad = jnp.pad(species.astype(jnp.int32), (0, _NPAD - _N))
    t2 = p['W_species'] @ p['lin0_W'] + p['lin0_b'][None, :]
    xi, h = _gather2_sc(p['W_species'], t2, sp_pad)
    dist2 = distances.reshape(1, _E)
    sw2 = switch.reshape(1, _E)
    boT = bond_order.T
    nblk_h = _EH // _BE
    for l in range(3):
        w0 = p[f'fc{l}_W0']
        w0p = (w0[:nfc].reshape(_RBDIM, _SUB, _NBONDS, _DIM)
               .transpose(0, 2, 1, 3).reshape(nfc, _DIM))
        hg = _gather_sc(h, edge_dst)
        msga = _edge_messages(dist2, sw2, boT, hg, w0p, 0, nblk_h)
        agg2a = _segment_sum_sc(msga, edge_src, 0)
        msgb = _edge_messages(dist2, sw2, boT, hg, w0p, nblk_h, nblk_h)
        agg2b = _segment_sum_sc(msgb, edge_src, _EH)
        nxt = (p[f'lin{l + 1}_W'], p[f'lin{l + 1}_b']) if l < 2 else (None,
                                                                      None)
        out = _node_post(xi, h, agg2a, agg2b, w0[nfc:], p[f'fc{l}_b0'],
                         p[f'fc{l}_W1'], p[f'fc{l}_b1'],
                         p[f'fc{l}_W2'], p[f'fc{l}_b2'], *nxt)
        if l < 2:
            xi, h = out
        else:
            xi = out[0]
    return xi[:_N]
